# Initial kernel scaffold; baseline (speedup 1.0000x reference)
import jax
import jax.numpy as jnp
from jax.experimental import pallas as pl


def kernel(char_ids, word_ids, char_table, word_table):
    raise NotImplementedError("write your pallas kernel here")


# speedup vs baseline: 10.1293x; 10.1293x over previous
import jax
import jax.numpy as jnp
from jax import lax
from jax.experimental import pallas as pl
from jax.experimental.pallas import tpu as pltpu
from jax.experimental.pallas import tpu_sc as plsc

NCHARS = 256
NWORDS = 100000
CHAR_EMB = 8
WORD_EMB = 16
W = 20
S = 50
B = 4096
N = B * S
ROWS_PER_POS = 2 + W
OUT_D = WORD_EMB + W * CHAR_EMB

NUM_WORKERS = 32
N_PER = N // NUM_WORKERS
CHUNK = 256
STEPS = N_PER // CHUNK
CROWS = CHUNK * ROWS_PER_POS


def _body(table_hbm, idx_hbm, out_hbm, idx_v, rows_v,
          sem_i, sem_g, sem_o0, sem_o1):
    ncores = 2
    worker = lax.axis_index("s") * ncores + lax.axis_index("c")
    wbase = worker * N_PER * ROWS_PER_POS

    def idx_copy(k):
        base = wbase + k * CROWS
        return pltpu.make_async_copy(idx_hbm.at[pl.ds(base, CROWS)],
                                     idx_v.at[k % 2], sem_i)

    def gather_copy(k):
        return pltpu.make_async_copy(table_hbm.at[idx_v.at[k % 2]],
                                     rows_v.at[k % 2], sem_g)

    def store_copy(k):
        base = wbase + k * CROWS
        sem = sem_o0 if k % 2 == 0 else sem_o1
        return pltpu.make_async_copy(rows_v.at[k % 2],
                                     out_hbm.at[pl.ds(base, CROWS), :], sem)

    c = idx_copy(0)
    c.start()
    c.wait()
    gather_copy(0).start()
    if STEPS > 1:
        idx_copy(1).start()
    for k in range(STEPS):
        gather_copy(k).wait()
        store_copy(k).start()
        if k + 1 < STEPS:
            idx_copy(k + 1).wait()
            if k >= 1:
                store_copy(k - 1).wait()
            gather_copy(k + 1).start()
            if k + 2 < STEPS:
                idx_copy(k + 2).start()
    if STEPS >= 2:
        store_copy(STEPS - 2).wait()
    store_copy(STEPS - 1).wait()


@jax.jit
def _run(table, idx_flat):
    mesh = plsc.VectorSubcoreMesh(core_axis_name="c", subcore_axis_name="s")
    return pl.kernel(
        _body,
        out_type=jax.ShapeDtypeStruct((N * ROWS_PER_POS, CHAR_EMB), jnp.float32),
        mesh=mesh,
        scratch_types=[
            pltpu.VMEM((2, CROWS), jnp.int32),
            pltpu.VMEM((2, CROWS, CHAR_EMB), jnp.float32),
            pltpu.SemaphoreType.DMA,
            pltpu.SemaphoreType.DMA,
            pltpu.SemaphoreType.DMA,
            pltpu.SemaphoreType.DMA,
        ],
        compiler_params=pltpu.CompilerParams(use_tc_tiling_on_sc=False),
    )(table, idx_flat)


def kernel(char_ids, word_ids, char_table, word_table):
    ct = char_table.at[0].set(0.0)
    wt = word_table.at[0].set(0.0)
    table = jnp.concatenate([ct, wt.reshape(-1, CHAR_EMB)], axis=0)
    wrow = 2 * word_ids + (NCHARS + 1)
    idx = jnp.concatenate(
        [wrow[..., None], wrow[..., None] + 1, char_ids], axis=-1)
    out = _run(table, idx.reshape(N * ROWS_PER_POS))
    return out.reshape(B, S, OUT_D)


# speedup vs baseline: 28.4781x; 2.8115x over previous
import jax
import jax.numpy as jnp
from jax import lax
from jax.experimental import pallas as pl
from jax.experimental.pallas import tpu as pltpu
from jax.experimental.pallas import tpu_sc as plsc

NCHARS = 256
NWORDS = 100000
CHAR_EMB = 8
WORD_EMB = 16
W = 20
S = 50
B = 4096
N = B * S
NPAIRS = (NCHARS + 1) * (NCHARS + 1)
ROWS_PER_POS = 1 + W // 2
OUT_D = WORD_EMB + W * CHAR_EMB

NUM_WORKERS = 32
N_PER = N // NUM_WORKERS
CHUNK = 256
STEPS = N_PER // CHUNK
CROWS = CHUNK * ROWS_PER_POS


def _body(table_hbm, idx_hbm, out_hbm, idx_v, rows_v,
          sem_i, sem_g, sem_o0, sem_o1):
    ncores = 2
    worker = lax.axis_index("s") * ncores + lax.axis_index("c")
    wbase = worker * N_PER * ROWS_PER_POS

    def idx_copy(k):
        base = wbase + k * CROWS
        return pltpu.make_async_copy(idx_hbm.at[pl.ds(base, CROWS)],
                                     idx_v.at[k % 2], sem_i)

    def gather_copy(k):
        return pltpu.make_async_copy(table_hbm.at[idx_v.at[k % 2]],
                                     rows_v.at[k % 2], sem_g)

    def store_copy(k):
        base = wbase + k * CROWS
        sem = sem_o0 if k % 2 == 0 else sem_o1
        return pltpu.make_async_copy(rows_v.at[k % 2],
                                     out_hbm.at[pl.ds(base, CROWS), :], sem)

    c = idx_copy(0)
    c.start()
    c.wait()
    gather_copy(0).start()
    if STEPS > 1:
        idx_copy(1).start()
    for k in range(STEPS):
        gather_copy(k).wait()
        store_copy(k).start()
        if k + 1 < STEPS:
            idx_copy(k + 1).wait()
            if k >= 1:
                store_copy(k - 1).wait()
            gather_copy(k + 1).start()
            if k + 2 < STEPS:
                idx_copy(k + 2).start()
    if STEPS >= 2:
        store_copy(STEPS - 2).wait()
    store_copy(STEPS - 1).wait()


@jax.jit
def _run(table, idx_flat):
    mesh = plsc.VectorSubcoreMesh(core_axis_name="c", subcore_axis_name="s")
    return pl.kernel(
        _body,
        out_type=jax.ShapeDtypeStruct((N * ROWS_PER_POS, WORD_EMB), jnp.float32),
        mesh=mesh,
        scratch_types=[
            pltpu.VMEM((2, CROWS), jnp.int32),
            pltpu.VMEM((2, CROWS, WORD_EMB), jnp.float32),
            pltpu.SemaphoreType.DMA,
            pltpu.SemaphoreType.DMA,
            pltpu.SemaphoreType.DMA,
            pltpu.SemaphoreType.DMA,
        ],
        compiler_params=pltpu.CompilerParams(use_tc_tiling_on_sc=False),
    )(table, idx_flat)


def kernel(char_ids, word_ids, char_table, word_table):
    ct = char_table.at[0].set(0.0)
    wt = word_table.at[0].set(0.0)
    nc = NCHARS + 1
    pair = jnp.concatenate(
        [jnp.broadcast_to(ct[:, None, :], (nc, nc, CHAR_EMB)),
         jnp.broadcast_to(ct[None, :, :], (nc, nc, CHAR_EMB))],
        axis=-1).reshape(NPAIRS, WORD_EMB)
    table = jnp.concatenate([pair, wt], axis=0)
    pid = char_ids[..., 0::2] * nc + char_ids[..., 1::2]
    wrow = word_ids[..., None] + NPAIRS
    idx = jnp.concatenate([wrow, pid], axis=-1)
    out = _run(table, idx.reshape(N * ROWS_PER_POS))
    return out.reshape(B, S, OUT_D)


# speedup vs baseline: 28.8620x; 1.0135x over previous
import jax
import jax.numpy as jnp
from jax import lax
from jax.experimental import pallas as pl
from jax.experimental.pallas import tpu as pltpu
from jax.experimental.pallas import tpu_sc as plsc

NCHARS = 256
NWORDS = 100000
CHAR_EMB = 8
WORD_EMB = 16
W = 20
S = 50
B = 4096
N = B * S
NC1 = NCHARS + 1
NPAIRS = NC1 * NC1
ROWS_PER_POS = 1 + W // 2
OUT_D = WORD_EMB + W * CHAR_EMB

NUM_WORKERS = 32
N_PER = N // NUM_WORKERS
CHUNK = 256
STEPS = N_PER // CHUNK
CROWS = CHUNK * ROWS_PER_POS
LANES = 16


def _body(table_hbm, cid_hbm, wid_hbm, out_hbm,
          cid_v, wid_v, idx_v, rows_v,
          sem_l, sem_g, sem_o0, sem_o1):
    ncores = 2
    worker = lax.axis_index("s") * ncores + lax.axis_index("c")
    wbase = worker * N_PER

    lane = lax.iota(jnp.int32, LANES)

    def loads(k):
        b = k % 2
        return (
            pltpu.make_async_copy(
                cid_hbm.at[pl.ds((wbase + k * CHUNK) * W, CHUNK * W)],
                cid_v.at[b], sem_l),
            pltpu.make_async_copy(
                wid_hbm.at[pl.ds(wbase + k * CHUNK, CHUNK)],
                wid_v.at[b], sem_l),
        )

    def loads_start(k):
        for c in loads(k):
            c.start()

    def loads_wait(k):
        for c in loads(k):
            c.wait()

    def build_idx(k):
        b = k % 2

        def it(i, _):
            p16 = i * LANES
            pv = p16 + lane
            wv = plsc.load_gather(wid_v.at[b], [pv])
            plsc.store_scatter(idx_v.at[b], [pv * ROWS_PER_POS],
                               wv + NPAIRS)
            cbase = pv * W
            dbase = pv * ROWS_PER_POS + 1
            for q in range(W // 2):
                c1 = plsc.load_gather(cid_v.at[b], [cbase + 2 * q])
                c2 = plsc.load_gather(cid_v.at[b], [cbase + 2 * q + 1])
                plsc.store_scatter(idx_v.at[b], [dbase + q],
                                   c1 * NC1 + c2)
            return ()

        lax.fori_loop(0, CHUNK // LANES, it, ())

    def gather_copy(k):
        return pltpu.make_async_copy(table_hbm.at[idx_v.at[k % 2]],
                                     rows_v.at[k % 2], sem_g)

    def store_copy(k):
        base = (wbase + k * CHUNK) * ROWS_PER_POS
        sem = sem_o0 if k % 2 == 0 else sem_o1
        return pltpu.make_async_copy(rows_v.at[k % 2],
                                     out_hbm.at[pl.ds(base, CROWS), :], sem)

    loads_start(0)
    if STEPS > 1:
        loads_start(1)
    loads_wait(0)
    build_idx(0)
    gather_copy(0).start()
    if STEPS > 1:
        loads_wait(1)
        build_idx(1)
    if STEPS > 2:
        loads_start(2)

    for k in range(STEPS):
        gather_copy(k).wait()
        store_copy(k).start()
        if k + 1 < STEPS:
            if k >= 1:
                store_copy(k - 1).wait()
            gather_copy(k + 1).start()
        if k + 2 < STEPS:
            loads_wait(k + 2)
            build_idx(k + 2)
        if k + 3 < STEPS:
            loads_start(k + 3)
    if STEPS >= 2:
        store_copy(STEPS - 2).wait()
    store_copy(STEPS - 1).wait()


@jax.jit
def _run(table, cid_flat, wid_flat):
    mesh = plsc.VectorSubcoreMesh(core_axis_name="c", subcore_axis_name="s")
    return pl.kernel(
        _body,
        out_type=jax.ShapeDtypeStruct((N * ROWS_PER_POS, WORD_EMB), jnp.float32),
        mesh=mesh,
        scratch_types=[
            pltpu.VMEM((2, CHUNK * W), jnp.int32),
            pltpu.VMEM((2, CHUNK), jnp.int32),
            pltpu.VMEM((2, CROWS), jnp.int32),
            pltpu.VMEM((2, CROWS, WORD_EMB), jnp.float32),
            pltpu.SemaphoreType.DMA,
            pltpu.SemaphoreType.DMA,
            pltpu.SemaphoreType.DMA,
            pltpu.SemaphoreType.DMA,
        ],
        compiler_params=pltpu.CompilerParams(use_tc_tiling_on_sc=False,
                                             needs_layout_passes=False),
    )(table, cid_flat, wid_flat)


def kernel(char_ids, word_ids, char_table, word_table):
    ct = char_table.at[0].set(0.0)
    wt = word_table.at[0].set(0.0)
    pair = jnp.concatenate(
        [jnp.broadcast_to(ct[:, None, :], (NC1, NC1, CHAR_EMB)),
         jnp.broadcast_to(ct[None, :, :], (NC1, NC1, CHAR_EMB))],
        axis=-1).reshape(NPAIRS, WORD_EMB)
    table = jnp.concatenate([pair, wt], axis=0)
    out = _run(table, char_ids.reshape(N * W), word_ids.reshape(N))
    return out.reshape(B, S, OUT_D)
